# baseline (speedup 1.0000x reference)
import jax
import jax.numpy as jnp
from jax import lax
from jax.experimental import pallas as pl
from jax.experimental.pallas import tpu as pltpu
from jax.experimental.pallas import tpu_sc as plsc

N = 10000
E = 320000
D = 128
H = 4
C = 32
NEG_SLOPE = 0.2
BN_EPS = 1e-5

_ROWS = 400
_NW = 32
_EP = 335872
_BAA = 256
_NBA = _EP // (_NW * _BAA)
_BC = 128
_NBC = _EP // (_NW * _BC)
_BB = 128
_NBB = _EP // (_NW * _BB)
_NPAD = N + 16
_NTAB = 10240
_TROWS = _NTAB // 16
_SC_MESH = dict(core_axis_name="c", subcore_axis_name="s")
_SC_PARAMS = pltpu.CompilerParams(needs_layout_passes=False)


def _pass_a_body(src_hbm, dst_hbm, xl_hbm, xr_hbm, att_hbm,
                 alpha_hbm, tmax_hbm,
                 att_v, sidx, didx, xlrows, xrrows, abuf, sem1, sem2):
    wid = lax.axis_index("s") * 2 + lax.axis_index("c")
    pltpu.sync_copy(att_hbm, att_v)
    lanes = lax.iota(jnp.int32, 16)
    rbase = lanes >> 2
    cbase = (lanes & 3) << 5

    def batch_body(b, mx):
        base = (wid * _NBA + b) * _BAA
        pltpu.sync_copy(src_hbm.at[pl.ds(base, _BAA)], sidx)
        pltpu.sync_copy(dst_hbm.at[pl.ds(base, _BAA)], didx)
        cp1 = pltpu.async_copy(xl_hbm.at[sidx], xlrows, sem1)
        cp2 = pltpu.async_copy(xr_hbm.at[didx], xrrows, sem2)
        cp1.wait()
        cp2.wait()

        def group_body(g, mx):
            row = rbase + g * 4
            acc = jnp.zeros((16,), jnp.float32)
            for c in range(C):
                col = cbase + c
                v = (plsc.load_gather(xlrows, [row, col])
                     + plsc.load_gather(xrrows, [row, col]))
                t = jnp.maximum(v, v * NEG_SLOPE)
                acc = acc + t * plsc.load_gather(att_v, [col])
            abuf[pl.ds(g * 16, 16)] = acc
            return jnp.maximum(mx, acc)

        mx = lax.fori_loop(0, _BAA // 4, group_body, mx)
        pltpu.sync_copy(abuf, alpha_hbm.at[pl.ds(base * 4, _BAA * 4)])
        return mx

    mx = lax.fori_loop(0, _NBA, batch_body,
                       jnp.full((16,), -3e38, jnp.float32))
    abuf[pl.ds(0, 16)] = mx
    pltpu.sync_copy(abuf.at[pl.ds(0, 16)], tmax_hbm.at[pl.ds(wid * 16, 16)])


def _pass_a(src, dst, xl, xr_pad, att_flat):
    return pl.kernel(
        _pass_a_body,
        out_type=[
            jax.ShapeDtypeStruct((_EP * 4,), jnp.float32),
            jax.ShapeDtypeStruct((_NW * 16,), jnp.float32),
        ],
        mesh=plsc.VectorSubcoreMesh(**_SC_MESH),
        compiler_params=_SC_PARAMS,
        scratch_types=[
            pltpu.VMEM((H * C,), jnp.float32),
            pltpu.VMEM((_BAA,), jnp.int32),
            pltpu.VMEM((_BAA,), jnp.int32),
            pltpu.VMEM((_BAA, D), jnp.float32),
            pltpu.VMEM((_BAA, D), jnp.float32),
            pltpu.VMEM((_BAA * 4,), jnp.float32),
            pltpu.SemaphoreType.DMA,
            pltpu.SemaphoreType.DMA,
        ],
    )(src, dst, xl, xr_pad, att_flat)


def _pass_b_body(dst_hbm, alpha_hbm, gmax_hbm, paytpl_hbm, zrows_hbm,
                 den_hbm,
                 gv, didx, abuf, pay, den_sh, sem):
    cid = lax.axis_index("c")
    sid = lax.axis_index("s")
    wid = sid * 2 + cid
    pltpu.sync_copy(gmax_hbm, gv)
    pltpu.sync_copy(paytpl_hbm, pay)
    pltpu.sync_copy(zrows_hbm, den_sh.at[pl.ds(sid * _TROWS, _TROWS)])
    plsc.subcore_barrier()
    lanes = lax.iota(jnp.int32, 16)
    rbase = lanes >> 2
    cols = lanes & 3
    g = gv[...]

    def batch_body(b, _):
        base = (wid * _NBB + b) * _BB
        pltpu.sync_copy(dst_hbm.at[pl.ds(base, _BB)], didx)
        pltpu.sync_copy(alpha_hbm.at[pl.ds(base * 4, _BB * 4)], abuf)
        for j in range(_BB // 4):
            a = abuf[pl.ds(j * 16, 16)]
            p = jnp.exp(a - g)
            plsc.store_scatter(pay, [rbase + 4 * j, cols], p)
        pltpu.sync_copy(pay, den_sh.at[didx], add=True)
        return 0

    lax.fori_loop(0, _NBB, batch_body, 0)
    plsc.subcore_barrier()
    pltpu.sync_copy(den_sh.at[pl.ds(sid * _TROWS, _TROWS)],
                    den_hbm.at[cid, pl.ds(sid * _TROWS, _TROWS)])


def _pass_b(dst, alpha_flat, gmax16, paytpl, zrows):
    return pl.kernel(
        _pass_b_body,
        out_type=[jax.ShapeDtypeStruct((2, _NTAB, D), jnp.float32)],
        mesh=plsc.VectorSubcoreMesh(**_SC_MESH),
        compiler_params=_SC_PARAMS,
        scratch_types=[
            pltpu.VMEM((16,), jnp.float32),
            pltpu.VMEM((_BB,), jnp.int32),
            pltpu.VMEM((_BB * 4,), jnp.float32),
            pltpu.VMEM((_BB, D), jnp.float32),
            pltpu.VMEM_SHARED((_NTAB, D), jnp.float32),
            pltpu.SemaphoreType.DMA,
        ],
    )(dst, alpha_flat, gmax16, paytpl, zrows)[0]


def _pass_c1_body(src_hbm, dst_hbm, alpha_hbm, gmax_hbm, xl_hbm, zrows_hbm,
                  acc_hbm,
                  gv, sidx, didx, abuf, wbuf, xlrows, contrib, acc_sh, sem):
    cid = lax.axis_index("c")
    sid = lax.axis_index("s")
    wid = sid * 2 + cid
    pltpu.sync_copy(gmax_hbm, gv)
    pltpu.sync_copy(zrows_hbm, acc_sh.at[pl.ds(sid * _TROWS, _TROWS)])
    plsc.subcore_barrier()
    g = gv[...]

    def batch_body(b, _):
        base = (wid * _NBC + b) * _BC
        pltpu.sync_copy(src_hbm.at[pl.ds(base, _BC)], sidx)
        pltpu.sync_copy(dst_hbm.at[pl.ds(base, _BC)], didx)
        pltpu.sync_copy(alpha_hbm.at[pl.ds(base * 4, _BC * 4)], abuf)
        cp = pltpu.async_copy(xl_hbm.at[sidx], xlrows, sem)
        for j in range(_BC // 4):
            a = abuf[pl.ds(j * 16, 16)]
            wbuf[pl.ds(j * 16, 16)] = jnp.exp(a - g)
        cp.wait()

        def edge_body(e, _):
            for jh in range(4):
                wsp = plsc.load_gather(
                    wbuf, [jnp.full((16,), e * 4 + jh, jnp.int32)])
                for k in range(2):
                    j = jh * 2 + k
                    xlv = xlrows[e, pl.ds(j * 16, 16)]
                    contrib[e, pl.ds(j * 16, 16)] = xlv * wsp
            return 0

        lax.fori_loop(0, _BC, edge_body, 0)
        pltpu.sync_copy(contrib, acc_sh.at[didx], add=True)
        return 0

    lax.fori_loop(0, _NBC, batch_body, 0)
    plsc.subcore_barrier()
    pltpu.sync_copy(acc_sh.at[pl.ds(sid * _TROWS, _TROWS)],
                    acc_hbm.at[cid, pl.ds(sid * _TROWS, _TROWS)])


def _pass_c1(src, dst, alpha_flat, gmax16, xl, zrows):
    return pl.kernel(
        _pass_c1_body,
        out_type=[jax.ShapeDtypeStruct((2, _NTAB, D), jnp.float32)],
        mesh=plsc.VectorSubcoreMesh(**_SC_MESH),
        compiler_params=_SC_PARAMS,
        scratch_types=[
            pltpu.VMEM((16,), jnp.float32),
            pltpu.VMEM((_BC,), jnp.int32),
            pltpu.VMEM((_BC,), jnp.int32),
            pltpu.VMEM((_BC * 4,), jnp.float32),
            pltpu.VMEM((_BC * 4,), jnp.float32),
            pltpu.VMEM((_BC, D), jnp.float32),
            pltpu.VMEM((_BC, D), jnp.float32),
            pltpu.VMEM_SHARED((_NTAB, D), jnp.float32),
            pltpu.SemaphoreType.DMA,
        ],
    )(src, dst, alpha_flat, gmax16, xl, zrows)[0]


def _pass_c2_body(src_hbm, dst_hbm, xws_hbm, zrows_hbm,
                  acc_hbm,
                  sidx, didx, xwrows, acc_sh, sem):
    cid = lax.axis_index("c")
    sid = lax.axis_index("s")
    wid = sid * 2 + cid
    pltpu.sync_copy(zrows_hbm, acc_sh.at[pl.ds(sid * _TROWS, _TROWS)])
    plsc.subcore_barrier()

    def batch_body(b, _):
        base = (wid * _NBA + b) * _BAA
        pltpu.sync_copy(src_hbm.at[pl.ds(base, _BAA)], sidx)
        pltpu.sync_copy(dst_hbm.at[pl.ds(base, _BAA)], didx)
        cp = pltpu.async_copy(xws_hbm.at[sidx], xwrows, sem)
        cp.wait()
        pltpu.sync_copy(xwrows, acc_sh.at[didx], add=True)
        return 0

    lax.fori_loop(0, _NBA, batch_body, 0)
    plsc.subcore_barrier()
    pltpu.sync_copy(acc_sh.at[pl.ds(sid * _TROWS, _TROWS)],
                    acc_hbm.at[cid, pl.ds(sid * _TROWS, _TROWS)])


def _pass_c2(src, dst, xws, zrows):
    return pl.kernel(
        _pass_c2_body,
        out_type=[jax.ShapeDtypeStruct((2, _NTAB, D), jnp.float32)],
        mesh=plsc.VectorSubcoreMesh(**_SC_MESH),
        compiler_params=_SC_PARAMS,
        scratch_types=[
            pltpu.VMEM((_BAA,), jnp.int32),
            pltpu.VMEM((_BAA,), jnp.int32),
            pltpu.VMEM((_BAA, D), jnp.float32),
            pltpu.VMEM_SHARED((_NTAB, D), jnp.float32),
            pltpu.SemaphoreType.DMA,
        ],
    )(src, dst, xws, zrows)[0]


def _wtab_body(d0_ref, d1_ref, xw_ref, w_ref, xws_ref):
    d = d0_ref[...] + d1_ref[...]
    invd = 1.0 / (d[:, :4] + 1e-16)
    deg = d[:, 4:5]
    dinv = jnp.where(deg > 0, jax.lax.rsqrt(jnp.maximum(deg, 1e-30)), 0.0)
    w_ref[...] = jnp.concatenate(
        [invd, dinv, jnp.zeros((d.shape[0], 11), jnp.float32)], axis=1)
    xws_ref[...] = xw_ref[...] * dinv


def _wtab(den, xw_pad):
    blk = 640
    row16 = pl.BlockSpec((blk, 16), lambda i: (i, 0))
    rowd = pl.BlockSpec((blk, D), lambda i: (i, 0))
    return pl.pallas_call(
        _wtab_body,
        grid=(_NTAB // blk,),
        in_specs=[pl.BlockSpec((blk, D), lambda i: (i, 0)),
                  pl.BlockSpec((blk, D), lambda i: (i, 0)),
                  rowd],
        out_specs=[row16, rowd],
        out_shape=[jax.ShapeDtypeStruct((_NTAB, 16), jnp.float32),
                   jax.ShapeDtypeStruct((_NTAB, D), jnp.float32)],
    )(den[0], den[1], xw_pad)


def _proj_body(x_ref, w_ref, xl_ref, xr_ref, xw_ref):
    y = jnp.dot(x_ref[...], w_ref[...], preferred_element_type=jnp.float32)
    xl_ref[...] = y[:, :D]
    xr_ref[...] = y[:, D:2 * D]
    xw_ref[...] = y[:, 2 * D:]


def _projections(x, W_l, W_r, W_gcn):
    wcat = jnp.concatenate([W_l, W_r, W_gcn], axis=1)
    grid = N // _ROWS
    row = pl.BlockSpec((_ROWS, D), lambda i: (i, 0))
    return pl.pallas_call(
        _proj_body,
        grid=(grid,),
        in_specs=[row, pl.BlockSpec((D, 3 * D), lambda i: (0, 0))],
        out_specs=[row, row, row],
        out_shape=[jax.ShapeDtypeStruct((N, D), jnp.float32)] * 3,
    )(x, wcat)


def _final_body(a0_ref, a1_ref, b0_ref, b1_ref, x_ref, w_ref,
                scale_ref, shift_ref, o_ref):
    rows = a0_ref.shape[0]
    w = w_ref[...]
    winv = jnp.reshape(
        jnp.broadcast_to(w[:, :4, None], (rows, 4, C)), (rows, D))
    dinv = w[:, 4:5]
    gat = (a0_ref[...] + a1_ref[...]) * winv
    gcn = (b0_ref[...] + b1_ref[...]) * dinv
    y = (gat + gcn + x_ref[...]) * scale_ref[...] + shift_ref[...]
    o_ref[...] = jnp.where(y > 0, y, jnp.exp(jnp.minimum(y, 0.0)) - 1.0)


def _final(acc_a, acc_b, x, wtab, scale, shift):
    grid = N // _ROWS
    row = pl.BlockSpec((_ROWS, D), lambda i: (i, 0))
    row16 = pl.BlockSpec((_ROWS, 16), lambda i: (i, 0))
    vec = pl.BlockSpec((1, D), lambda i: (0, 0))
    return pl.pallas_call(
        _final_body,
        grid=(grid,),
        in_specs=[row, row, row, row, row, row16, vec, vec],
        out_specs=row,
        out_shape=jax.ShapeDtypeStruct((N, D), jnp.float32),
    )(acc_a[0, :N], acc_a[1, :N], acc_b[0, :N], acc_b[1, :N], x,
      wtab[:N], scale.reshape(1, D), shift.reshape(1, D))


def kernel(x, edge_index, W_l, W_r, att, b_gat, W_gcn, b_gcn, gamma, beta):
    ne = E + N
    loops = jnp.arange(N, dtype=jnp.int32)
    src = jnp.concatenate([edge_index[0].astype(jnp.int32), loops,
                           jnp.zeros((_EP - ne,), jnp.int32)])
    dst = jnp.concatenate([edge_index[1].astype(jnp.int32), loops,
                           jnp.full((_EP - ne,), N, jnp.int32)])

    xl, xr, xw = _projections(x, W_l, W_r, W_gcn)
    zpad = jnp.zeros((_NPAD - N, D), jnp.float32)
    xr_pad = jnp.concatenate([xr, zpad])

    alpha_flat, tmax = _pass_a(src, dst, xl, xr_pad, att.reshape(H * C))
    gmax16 = jnp.full((16,), jnp.max(tmax), jnp.float32)

    paytpl = jnp.zeros((_BB, D), jnp.float32).at[:, 4].set(1.0)
    zrows = jnp.zeros((_TROWS, D), jnp.float32)
    den = _pass_b(dst, alpha_flat, gmax16, paytpl, zrows)

    xw_tab = jnp.concatenate([xw, jnp.zeros((_NTAB - N, D), jnp.float32)])
    wtab, xws_tab = _wtab(den, xw_tab)
    xws = xws_tab[:_NPAD]

    acc_a = _pass_c1(src, dst, alpha_flat, gmax16, xl, zrows)
    acc_b = _pass_c2(src, dst, xws, zrows)

    scale = gamma * jax.lax.rsqrt(jnp.float32(1.0 + BN_EPS))
    shift = (b_gat + b_gcn) * scale + beta
    return _final(acc_a, acc_b, x, wtab, scale, shift)


# speedup vs baseline: 1.1548x; 1.1548x over previous
import jax
import jax.numpy as jnp
from jax import lax
from jax.experimental import pallas as pl
from jax.experimental.pallas import tpu as pltpu
from jax.experimental.pallas import tpu_sc as plsc

N = 10000
E = 320000
D = 128
H = 4
C = 32
NEG_SLOPE = 0.2
BN_EPS = 1e-5

_ROWS = 400
_NW = 32
_EP = 331776
_BAA = 128
_NBA = _EP // (_NW * _BAA)
_BC = 128
_NBC = _EP // (_NW * _BC)
_BB = 128
_NBB = _EP // (_NW * _BB)
_NPAD = N + 16
_NTAB = 10240
_TROWS = _NTAB // 16
_SC_MESH = dict(core_axis_name="c", subcore_axis_name="s")
_SC_PARAMS = pltpu.CompilerParams(needs_layout_passes=False)


def _pass_a_body(src_hbm, dst_hbm, xl_hbm, xr_hbm, att_hbm,
                 alpha_hbm, tmax_hbm,
                 att_v, sidx, didx, xlrows, xrrows, abuf, sem1, sem2):
    wid = lax.axis_index("s") * 2 + lax.axis_index("c")
    pltpu.sync_copy(att_hbm, att_v)
    lanes = lax.iota(jnp.int32, 16)
    rbase = lanes >> 2
    cbase = (lanes & 3) << 5

    def batch_body(b, mx):
        base = (wid * _NBA + b) * _BAA
        pltpu.sync_copy(src_hbm.at[pl.ds(base, _BAA)], sidx)
        pltpu.sync_copy(dst_hbm.at[pl.ds(base, _BAA)], didx)
        cp1 = pltpu.async_copy(xl_hbm.at[sidx], xlrows, sem1)
        cp2 = pltpu.async_copy(xr_hbm.at[didx], xrrows, sem2)
        cp1.wait()
        cp2.wait()

        def group_body(g, mx):
            row = rbase + g * 4
            acc = jnp.zeros((16,), jnp.float32)
            for c in range(C):
                col = cbase + c
                v = (plsc.load_gather(xlrows, [row, col])
                     + plsc.load_gather(xrrows, [row, col]))
                t = jnp.maximum(v, v * NEG_SLOPE)
                acc = acc + t * plsc.load_gather(att_v, [col])
            abuf[pl.ds(g * 16, 16)] = acc
            return jnp.maximum(mx, acc)

        mx = lax.fori_loop(0, _BAA // 4, group_body, mx)
        pltpu.sync_copy(abuf, alpha_hbm.at[pl.ds(base * 4, _BAA * 4)])
        return mx

    mx = lax.fori_loop(0, _NBA, batch_body,
                       jnp.full((16,), -3e38, jnp.float32))
    abuf[pl.ds(0, 16)] = mx
    pltpu.sync_copy(abuf.at[pl.ds(0, 16)], tmax_hbm.at[pl.ds(wid * 16, 16)])


def _pass_a(src, dst, xl, xr_pad, att_flat):
    return pl.kernel(
        _pass_a_body,
        out_type=[
            jax.ShapeDtypeStruct((_EP * 4,), jnp.float32),
            jax.ShapeDtypeStruct((_NW * 16,), jnp.float32),
        ],
        mesh=plsc.VectorSubcoreMesh(**_SC_MESH),
        compiler_params=_SC_PARAMS,
        scratch_types=[
            pltpu.VMEM((H * C,), jnp.float32),
            pltpu.VMEM((_BAA,), jnp.int32),
            pltpu.VMEM((_BAA,), jnp.int32),
            pltpu.VMEM((_BAA, D), jnp.float32),
            pltpu.VMEM((_BAA, D), jnp.float32),
            pltpu.VMEM((_BAA * 4,), jnp.float32),
            pltpu.SemaphoreType.DMA,
            pltpu.SemaphoreType.DMA,
        ],
    )(src, dst, xl, xr_pad, att_flat)


def _pass_b_body(dst_hbm, alpha_hbm, gmax_hbm, paytpl_hbm, zrows_hbm,
                 den_hbm,
                 gv, didx, abuf, pay, den_sh, sem):
    cid = lax.axis_index("c")
    sid = lax.axis_index("s")
    wid = sid * 2 + cid
    pltpu.sync_copy(gmax_hbm, gv)
    pltpu.sync_copy(paytpl_hbm, pay)
    pltpu.sync_copy(zrows_hbm, den_sh.at[pl.ds(sid * _TROWS, _TROWS)])
    plsc.subcore_barrier()
    lanes = lax.iota(jnp.int32, 16)
    rbase = lanes >> 2
    cols = lanes & 3
    g = gv[...]

    def batch_body(b, _):
        base = (wid * _NBB + b) * _BB
        pltpu.sync_copy(dst_hbm.at[pl.ds(base, _BB)], didx)
        pltpu.sync_copy(alpha_hbm.at[pl.ds(base * 4, _BB * 4)], abuf)
        for j in range(_BB // 4):
            a = abuf[pl.ds(j * 16, 16)]
            p = jnp.exp(a - g)
            plsc.store_scatter(pay, [rbase + 4 * j, cols], p)
        pltpu.sync_copy(pay, den_sh.at[didx], add=True)
        return 0

    lax.fori_loop(0, _NBB, batch_body, 0)
    plsc.subcore_barrier()
    pltpu.sync_copy(den_sh.at[pl.ds(sid * _TROWS, _TROWS)],
                    den_hbm.at[cid, pl.ds(sid * _TROWS, _TROWS)])


def _pass_b(dst, alpha_flat, gmax16, paytpl, zrows):
    return pl.kernel(
        _pass_b_body,
        out_type=[jax.ShapeDtypeStruct((2, _NTAB, D), jnp.float32)],
        mesh=plsc.VectorSubcoreMesh(**_SC_MESH),
        compiler_params=_SC_PARAMS,
        scratch_types=[
            pltpu.VMEM((16,), jnp.float32),
            pltpu.VMEM((_BB,), jnp.int32),
            pltpu.VMEM((_BB * 4,), jnp.float32),
            pltpu.VMEM((_BB, D), jnp.float32),
            pltpu.VMEM_SHARED((_NTAB, D), jnp.float32),
            pltpu.SemaphoreType.DMA,
        ],
    )(dst, alpha_flat, gmax16, paytpl, zrows)[0]


def _pass_c1_body(src_hbm, dst_hbm, alpha_hbm, gmax_hbm, xl_hbm, zrows_hbm,
                  acc_hbm,
                  gv, sidx, didx, abuf, wbuf, xlrows, contrib, acc_sh, sem):
    cid = lax.axis_index("c")
    sid = lax.axis_index("s")
    wid = sid * 2 + cid
    pltpu.sync_copy(gmax_hbm, gv)
    pltpu.sync_copy(zrows_hbm, acc_sh.at[pl.ds(sid * _TROWS, _TROWS)])
    plsc.subcore_barrier()
    g = gv[...]

    def batch_body(b, _):
        base = (wid * _NBC + b) * _BC
        pltpu.sync_copy(src_hbm.at[pl.ds(base, _BC)], sidx)
        pltpu.sync_copy(dst_hbm.at[pl.ds(base, _BC)], didx)
        pltpu.sync_copy(alpha_hbm.at[pl.ds(base * 4, _BC * 4)], abuf)
        cp = pltpu.async_copy(xl_hbm.at[sidx], xlrows, sem)
        for j in range(_BC // 4):
            a = abuf[pl.ds(j * 16, 16)]
            wbuf[pl.ds(j * 16, 16)] = jnp.exp(a - g)
        cp.wait()

        def edge_body(e, _):
            for jh in range(4):
                wsp = plsc.load_gather(
                    wbuf, [jnp.full((16,), e * 4 + jh, jnp.int32)])
                for k in range(2):
                    j = jh * 2 + k
                    xlv = xlrows[e, pl.ds(j * 16, 16)]
                    contrib[e, pl.ds(j * 16, 16)] = xlv * wsp
            return 0

        lax.fori_loop(0, _BC, edge_body, 0)
        pltpu.sync_copy(contrib, acc_sh.at[didx], add=True)
        return 0

    lax.fori_loop(0, _NBC, batch_body, 0)
    plsc.subcore_barrier()
    pltpu.sync_copy(acc_sh.at[pl.ds(sid * _TROWS, _TROWS)],
                    acc_hbm.at[cid, pl.ds(sid * _TROWS, _TROWS)])


def _pass_c1(src, dst, alpha_flat, gmax16, xl, zrows):
    return pl.kernel(
        _pass_c1_body,
        out_type=[jax.ShapeDtypeStruct((2, _NTAB, D), jnp.float32)],
        mesh=plsc.VectorSubcoreMesh(**_SC_MESH),
        compiler_params=_SC_PARAMS,
        scratch_types=[
            pltpu.VMEM((16,), jnp.float32),
            pltpu.VMEM((_BC,), jnp.int32),
            pltpu.VMEM((_BC,), jnp.int32),
            pltpu.VMEM((_BC * 4,), jnp.float32),
            pltpu.VMEM((_BC * 4,), jnp.float32),
            pltpu.VMEM((_BC, D), jnp.float32),
            pltpu.VMEM((_BC, D), jnp.float32),
            pltpu.VMEM_SHARED((_NTAB, D), jnp.float32),
            pltpu.SemaphoreType.DMA,
        ],
    )(src, dst, alpha_flat, gmax16, xl, zrows)[0]


def _pass_c2_body(src_hbm, dst_hbm, xws_hbm, zrows_hbm,
                  acc_hbm,
                  sidx, didx, xwrows, acc_sh, sem):
    cid = lax.axis_index("c")
    sid = lax.axis_index("s")
    wid = sid * 2 + cid
    pltpu.sync_copy(zrows_hbm, acc_sh.at[pl.ds(sid * _TROWS, _TROWS)])
    plsc.subcore_barrier()

    def batch_body(b, _):
        base = (wid * _NBA + b) * _BAA
        pltpu.sync_copy(src_hbm.at[pl.ds(base, _BAA)], sidx)
        pltpu.sync_copy(dst_hbm.at[pl.ds(base, _BAA)], didx)
        cp = pltpu.async_copy(xws_hbm.at[sidx], xwrows, sem)
        cp.wait()
        pltpu.sync_copy(xwrows, acc_sh.at[didx], add=True)
        return 0

    lax.fori_loop(0, _NBA, batch_body, 0)
    plsc.subcore_barrier()
    pltpu.sync_copy(acc_sh.at[pl.ds(sid * _TROWS, _TROWS)],
                    acc_hbm.at[cid, pl.ds(sid * _TROWS, _TROWS)])


def _pass_c2(src, dst, xws, zrows):
    return pl.kernel(
        _pass_c2_body,
        out_type=[jax.ShapeDtypeStruct((2, _NTAB, D), jnp.float32)],
        mesh=plsc.VectorSubcoreMesh(**_SC_MESH),
        compiler_params=_SC_PARAMS,
        scratch_types=[
            pltpu.VMEM((_BAA,), jnp.int32),
            pltpu.VMEM((_BAA,), jnp.int32),
            pltpu.VMEM((_BAA, D), jnp.float32),
            pltpu.VMEM_SHARED((_NTAB, D), jnp.float32),
            pltpu.SemaphoreType.DMA,
        ],
    )(src, dst, xws, zrows)[0]


def _wtab_body(d0_ref, d1_ref, xw_ref, w_ref, xws_ref):
    d = d0_ref[...] + d1_ref[...]
    invd = 1.0 / (d[:, :4] + 1e-16)
    deg = d[:, 4:5]
    dinv = jnp.where(deg > 0, jax.lax.rsqrt(jnp.maximum(deg, 1e-30)), 0.0)
    w_ref[...] = jnp.concatenate(
        [invd, dinv, jnp.zeros((d.shape[0], 11), jnp.float32)], axis=1)
    xws_ref[...] = xw_ref[...] * dinv


def _wtab(den, xw_pad):
    blk = 640
    row16 = pl.BlockSpec((blk, 16), lambda i: (i, 0))
    rowd = pl.BlockSpec((blk, D), lambda i: (i, 0))
    return pl.pallas_call(
        _wtab_body,
        grid=(_NTAB // blk,),
        in_specs=[pl.BlockSpec((blk, D), lambda i: (i, 0)),
                  pl.BlockSpec((blk, D), lambda i: (i, 0)),
                  rowd],
        out_specs=[row16, rowd],
        out_shape=[jax.ShapeDtypeStruct((_NTAB, 16), jnp.float32),
                   jax.ShapeDtypeStruct((_NTAB, D), jnp.float32)],
    )(den[0], den[1], xw_pad)


def _proj_body(x_ref, w_ref, xl_ref, xr_ref, xw_ref):
    y = jnp.dot(x_ref[...], w_ref[...], preferred_element_type=jnp.float32)
    xl_ref[...] = y[:, :D]
    xr_ref[...] = y[:, D:2 * D]
    xw_ref[...] = y[:, 2 * D:]


def _projections(x, W_l, W_r, W_gcn):
    wcat = jnp.concatenate([W_l, W_r, W_gcn], axis=1)
    grid = N // _ROWS
    row = pl.BlockSpec((_ROWS, D), lambda i: (i, 0))
    return pl.pallas_call(
        _proj_body,
        grid=(grid,),
        in_specs=[row, pl.BlockSpec((D, 3 * D), lambda i: (0, 0))],
        out_specs=[row, row, row],
        out_shape=[jax.ShapeDtypeStruct((N, D), jnp.float32)] * 3,
    )(x, wcat)


def _final_body(a0_ref, a1_ref, b0_ref, b1_ref, x_ref, w_ref,
                scale_ref, shift_ref, o_ref):
    rows = a0_ref.shape[0]
    w = w_ref[...]
    winv = jnp.reshape(
        jnp.broadcast_to(w[:, :4, None], (rows, 4, C)), (rows, D))
    dinv = w[:, 4:5]
    gat = (a0_ref[...] + a1_ref[...]) * winv
    gcn = (b0_ref[...] + b1_ref[...]) * dinv
    y = (gat + gcn + x_ref[...]) * scale_ref[...] + shift_ref[...]
    o_ref[...] = jnp.where(y > 0, y, jnp.exp(jnp.minimum(y, 0.0)) - 1.0)


def _final(acc_a, acc_b, x, wtab, scale, shift):
    grid = N // _ROWS
    row = pl.BlockSpec((_ROWS, D), lambda i: (i, 0))
    row16 = pl.BlockSpec((_ROWS, 16), lambda i: (i, 0))
    vec = pl.BlockSpec((1, D), lambda i: (0, 0))
    return pl.pallas_call(
        _final_body,
        grid=(grid,),
        in_specs=[row, row, row, row, row, row16, vec, vec],
        out_specs=row,
        out_shape=jax.ShapeDtypeStruct((N, D), jnp.float32),
    )(acc_a[0, :N], acc_a[1, :N], acc_b[0, :N], acc_b[1, :N], x,
      wtab[:N], scale.reshape(1, D), shift.reshape(1, D))


def kernel(x, edge_index, W_l, W_r, att, b_gat, W_gcn, b_gcn, gamma, beta):
    ne = E + N
    loops = jnp.arange(N, dtype=jnp.int32)
    src = jnp.concatenate([edge_index[0].astype(jnp.int32), loops,
                           jnp.zeros((_EP - ne,), jnp.int32)])
    dst = jnp.concatenate([edge_index[1].astype(jnp.int32), loops,
                           jnp.full((_EP - ne,), N, jnp.int32)])

    xl, xr, xw = _projections(x, W_l, W_r, W_gcn)
    zpad = jnp.zeros((_NPAD - N, D), jnp.float32)
    xr_pad = jnp.concatenate([xr, zpad])

    alpha_flat, tmax = _pass_a(src, dst, xl, xr_pad, att.reshape(H * C))
    gmax16 = jnp.full((16,), jnp.max(tmax), jnp.float32)

    paytpl = jnp.zeros((_BB, D), jnp.float32).at[:, 4].set(1.0)
    zrows = jnp.zeros((_TROWS, D), jnp.float32)
    den = _pass_b(dst, alpha_flat, gmax16, paytpl, zrows)

    xw_tab = jnp.concatenate([xw, jnp.zeros((_NTAB - N, D), jnp.float32)])
    wtab, xws_tab = _wtab(den, xw_tab)
    xws = xws_tab[:_NPAD]

    acc_a = _pass_c1(src, dst, alpha_flat, gmax16, xl, zrows)
    acc_b = _pass_c2(src, dst, xws, zrows)

    scale = gamma * jax.lax.rsqrt(jnp.float32(1.0 + BN_EPS))
    shift = (b_gat + b_gcn) * scale + beta
    return _final(acc_a, acc_b, x, wtab, scale, shift)


# speedup vs baseline: 1.2617x; 1.0925x over previous
import jax
import jax.numpy as jnp
from jax import lax
from jax.experimental import pallas as pl
from jax.experimental.pallas import tpu as pltpu
from jax.experimental.pallas import tpu_sc as plsc

N = 10000
E = 320000
D = 128
H = 4
C = 32
NEG_SLOPE = 0.2
BN_EPS = 1e-5

_ROWS = 400
_NW = 32
_EP = 331776
_BAA = 128
_NBA = _EP // (_NW * _BAA)
_BC = 128
_NBC = _EP // (_NW * _BC)
_BB = 128
_NBB = _EP // (_NW * _BB)
_NPAD = N + 16
_NTAB = 10240
_TROWS = _NTAB // 16
_SC_MESH = dict(core_axis_name="c", subcore_axis_name="s")
_SC_PARAMS = pltpu.CompilerParams(needs_layout_passes=False)


def _pass_a_body(src_hbm, dst_hbm, xl_hbm, xr_hbm, att_hbm,
                 alpha_hbm, tmax_hbm,
                 att_v, sidx, didx, xlrows, xrrows, abuf, sem1, sem2):
    wid = lax.axis_index("s") * 2 + lax.axis_index("c")
    pltpu.sync_copy(att_hbm, att_v)
    lanes = lax.iota(jnp.int32, 16)
    rbase = lanes >> 2
    cbase = (lanes & 3) << 5
    att_c = [plsc.load_gather(att_v, [cbase + c]) for c in range(C)]

    def batch_body(b, mx):
        base = (wid * _NBA + b) * _BAA
        pltpu.sync_copy(src_hbm.at[pl.ds(base, _BAA)], sidx)
        pltpu.sync_copy(dst_hbm.at[pl.ds(base, _BAA)], didx)
        cp1 = pltpu.async_copy(xl_hbm.at[sidx], xlrows, sem1)
        cp2 = pltpu.async_copy(xr_hbm.at[didx], xrrows, sem2)
        cp1.wait()
        cp2.wait()

        def group_body(g, mx):
            row = rbase + g * 4
            acc = jnp.zeros((16,), jnp.float32)
            for c in range(C):
                col = cbase + c
                v = (plsc.load_gather(xlrows, [row, col])
                     + plsc.load_gather(xrrows, [row, col]))
                t = jnp.maximum(v, v * NEG_SLOPE)
                acc = acc + t * att_c[c]
            abuf[pl.ds(g * 16, 16)] = acc
            return jnp.maximum(mx, acc)

        mx = lax.fori_loop(0, _BAA // 4, group_body, mx)
        pltpu.sync_copy(abuf, alpha_hbm.at[pl.ds(base * 4, _BAA * 4)])
        return mx

    mx = lax.fori_loop(0, _NBA, batch_body,
                       jnp.full((16,), -3e38, jnp.float32))
    abuf[pl.ds(0, 16)] = mx
    pltpu.sync_copy(abuf.at[pl.ds(0, 16)], tmax_hbm.at[pl.ds(wid * 16, 16)])


def _pass_a(src, dst, xl, xr_pad, att_flat):
    return pl.kernel(
        _pass_a_body,
        out_type=[
            jax.ShapeDtypeStruct((_EP * 4,), jnp.float32),
            jax.ShapeDtypeStruct((_NW * 16,), jnp.float32),
        ],
        mesh=plsc.VectorSubcoreMesh(**_SC_MESH),
        compiler_params=_SC_PARAMS,
        scratch_types=[
            pltpu.VMEM((H * C,), jnp.float32),
            pltpu.VMEM((_BAA,), jnp.int32),
            pltpu.VMEM((_BAA,), jnp.int32),
            pltpu.VMEM((_BAA, D), jnp.float32),
            pltpu.VMEM((_BAA, D), jnp.float32),
            pltpu.VMEM((_BAA * 4,), jnp.float32),
            pltpu.SemaphoreType.DMA,
            pltpu.SemaphoreType.DMA,
        ],
    )(src, dst, xl, xr_pad, att_flat)


def _pass_b_body(dst_hbm, alpha_hbm, gmax_hbm, paytpl_hbm, zrows_hbm,
                 den_hbm,
                 gv, didx, abuf, pay, den_sh, sem):
    cid = lax.axis_index("c")
    sid = lax.axis_index("s")
    wid = sid * 2 + cid
    pltpu.sync_copy(gmax_hbm, gv)
    pltpu.sync_copy(paytpl_hbm, pay)
    pltpu.sync_copy(zrows_hbm, den_sh.at[pl.ds(sid * _TROWS, _TROWS)])
    plsc.subcore_barrier()
    lanes = lax.iota(jnp.int32, 16)
    rbase = lanes >> 2
    cols = lanes & 3
    g = gv[...]

    def batch_body(b, _):
        base = (wid * _NBB + b) * _BB
        pltpu.sync_copy(dst_hbm.at[pl.ds(base, _BB)], didx)
        pltpu.sync_copy(alpha_hbm.at[pl.ds(base * 4, _BB * 4)], abuf)
        for j in range(_BB // 4):
            a = abuf[pl.ds(j * 16, 16)]
            p = jnp.exp(a - g)
            plsc.store_scatter(pay, [rbase + 4 * j, cols], p)
        pltpu.sync_copy(pay, den_sh.at[didx], add=True)
        return 0

    lax.fori_loop(0, _NBB, batch_body, 0)
    plsc.subcore_barrier()
    pltpu.sync_copy(den_sh.at[pl.ds(sid * _TROWS, _TROWS)],
                    den_hbm.at[cid, pl.ds(sid * _TROWS, _TROWS)])


def _pass_b(dst, alpha_flat, gmax16, paytpl, zrows):
    return pl.kernel(
        _pass_b_body,
        out_type=[jax.ShapeDtypeStruct((2, _NTAB, D), jnp.float32)],
        mesh=plsc.VectorSubcoreMesh(**_SC_MESH),
        compiler_params=_SC_PARAMS,
        scratch_types=[
            pltpu.VMEM((16,), jnp.float32),
            pltpu.VMEM((_BB,), jnp.int32),
            pltpu.VMEM((_BB * 4,), jnp.float32),
            pltpu.VMEM((_BB, D), jnp.float32),
            pltpu.VMEM_SHARED((_NTAB, D), jnp.float32),
            pltpu.SemaphoreType.DMA,
        ],
    )(dst, alpha_flat, gmax16, paytpl, zrows)[0]


def _pass_c1_body(src_hbm, dst_hbm, alpha_hbm, gmax_hbm, xl_hbm, zrows_hbm,
                  acc_hbm,
                  gv, sidx, didx, abuf, wbuf, xlrows, contrib, acc_sh, sem):
    cid = lax.axis_index("c")
    sid = lax.axis_index("s")
    wid = sid * 2 + cid
    pltpu.sync_copy(gmax_hbm, gv)
    pltpu.sync_copy(zrows_hbm, acc_sh.at[pl.ds(sid * _TROWS, _TROWS)])
    plsc.subcore_barrier()
    g = gv[...]

    def batch_body(b, _):
        base = (wid * _NBC + b) * _BC
        pltpu.sync_copy(src_hbm.at[pl.ds(base, _BC)], sidx)
        pltpu.sync_copy(dst_hbm.at[pl.ds(base, _BC)], didx)
        pltpu.sync_copy(alpha_hbm.at[pl.ds(base * 4, _BC * 4)], abuf)
        cp = pltpu.async_copy(xl_hbm.at[sidx], xlrows, sem)
        for j in range(_BC // 4):
            a = abuf[pl.ds(j * 16, 16)]
            wbuf[pl.ds(j * 16, 16)] = jnp.exp(a - g)
        cp.wait()

        def edge_body(e, _):
            for jh in range(4):
                wsp = plsc.load_gather(
                    wbuf, [jnp.full((16,), e * 4 + jh, jnp.int32)])
                for k in range(2):
                    j = jh * 2 + k
                    xlv = xlrows[e, pl.ds(j * 16, 16)]
                    contrib[e, pl.ds(j * 16, 16)] = xlv * wsp
            return 0

        lax.fori_loop(0, _BC, edge_body, 0)
        pltpu.sync_copy(contrib, acc_sh.at[didx], add=True)
        return 0

    lax.fori_loop(0, _NBC, batch_body, 0)
    plsc.subcore_barrier()
    pltpu.sync_copy(acc_sh.at[pl.ds(sid * _TROWS, _TROWS)],
                    acc_hbm.at[cid, pl.ds(sid * _TROWS, _TROWS)])


def _pass_c1(src, dst, alpha_flat, gmax16, xl, zrows):
    return pl.kernel(
        _pass_c1_body,
        out_type=[jax.ShapeDtypeStruct((2, _NTAB, D), jnp.float32)],
        mesh=plsc.VectorSubcoreMesh(**_SC_MESH),
        compiler_params=_SC_PARAMS,
        scratch_types=[
            pltpu.VMEM((16,), jnp.float32),
            pltpu.VMEM((_BC,), jnp.int32),
            pltpu.VMEM((_BC,), jnp.int32),
            pltpu.VMEM((_BC * 4,), jnp.float32),
            pltpu.VMEM((_BC * 4,), jnp.float32),
            pltpu.VMEM((_BC, D), jnp.float32),
            pltpu.VMEM((_BC, D), jnp.float32),
            pltpu.VMEM_SHARED((_NTAB, D), jnp.float32),
            pltpu.SemaphoreType.DMA,
        ],
    )(src, dst, alpha_flat, gmax16, xl, zrows)[0]


def _pass_c2_body(src_hbm, dst_hbm, xws_hbm, zrows_hbm,
                  acc_hbm,
                  sidx, didx, xwrows, acc_sh, sem):
    cid = lax.axis_index("c")
    sid = lax.axis_index("s")
    wid = sid * 2 + cid
    pltpu.sync_copy(zrows_hbm, acc_sh.at[pl.ds(sid * _TROWS, _TROWS)])
    plsc.subcore_barrier()

    def batch_body(b, _):
        base = (wid * _NBA + b) * _BAA
        pltpu.sync_copy(src_hbm.at[pl.ds(base, _BAA)], sidx)
        pltpu.sync_copy(dst_hbm.at[pl.ds(base, _BAA)], didx)
        cp = pltpu.async_copy(xws_hbm.at[sidx], xwrows, sem)
        cp.wait()
        pltpu.sync_copy(xwrows, acc_sh.at[didx], add=True)
        return 0

    lax.fori_loop(0, _NBA, batch_body, 0)
    plsc.subcore_barrier()
    pltpu.sync_copy(acc_sh.at[pl.ds(sid * _TROWS, _TROWS)],
                    acc_hbm.at[cid, pl.ds(sid * _TROWS, _TROWS)])


def _pass_c2(src, dst, xws, zrows):
    return pl.kernel(
        _pass_c2_body,
        out_type=[jax.ShapeDtypeStruct((2, _NTAB, D), jnp.float32)],
        mesh=plsc.VectorSubcoreMesh(**_SC_MESH),
        compiler_params=_SC_PARAMS,
        scratch_types=[
            pltpu.VMEM((_BAA,), jnp.int32),
            pltpu.VMEM((_BAA,), jnp.int32),
            pltpu.VMEM((_BAA, D), jnp.float32),
            pltpu.VMEM_SHARED((_NTAB, D), jnp.float32),
            pltpu.SemaphoreType.DMA,
        ],
    )(src, dst, xws, zrows)[0]


def _wtab_body(d0_ref, d1_ref, xw_ref, w_ref, xws_ref):
    d = d0_ref[...] + d1_ref[...]
    invd = 1.0 / (d[:, :4] + 1e-16)
    deg = d[:, 4:5]
    dinv = jnp.where(deg > 0, jax.lax.rsqrt(jnp.maximum(deg, 1e-30)), 0.0)
    w_ref[...] = jnp.concatenate(
        [invd, dinv, jnp.zeros((d.shape[0], 11), jnp.float32)], axis=1)
    xws_ref[...] = xw_ref[...] * dinv


def _wtab(den, xw_pad):
    blk = 640
    row16 = pl.BlockSpec((blk, 16), lambda i: (i, 0))
    rowd = pl.BlockSpec((blk, D), lambda i: (i, 0))
    return pl.pallas_call(
        _wtab_body,
        grid=(_NTAB // blk,),
        in_specs=[pl.BlockSpec((blk, D), lambda i: (i, 0)),
                  pl.BlockSpec((blk, D), lambda i: (i, 0)),
                  rowd],
        out_specs=[row16, rowd],
        out_shape=[jax.ShapeDtypeStruct((_NTAB, 16), jnp.float32),
                   jax.ShapeDtypeStruct((_NTAB, D), jnp.float32)],
    )(den[0], den[1], xw_pad)


def _proj_body(x_ref, w_ref, xl_ref, xr_ref, xw_ref):
    y = jnp.dot(x_ref[...], w_ref[...], preferred_element_type=jnp.float32)
    xl_ref[...] = y[:, :D]
    xr_ref[...] = y[:, D:2 * D]
    xw_ref[...] = y[:, 2 * D:]


def _projections(x, W_l, W_r, W_gcn):
    wcat = jnp.concatenate([W_l, W_r, W_gcn], axis=1)
    grid = N // _ROWS
    row = pl.BlockSpec((_ROWS, D), lambda i: (i, 0))
    return pl.pallas_call(
        _proj_body,
        grid=(grid,),
        in_specs=[row, pl.BlockSpec((D, 3 * D), lambda i: (0, 0))],
        out_specs=[row, row, row],
        out_shape=[jax.ShapeDtypeStruct((N, D), jnp.float32)] * 3,
    )(x, wcat)


def _final_body(a0_ref, a1_ref, b0_ref, b1_ref, x_ref, w_ref,
                scale_ref, shift_ref, o_ref):
    rows = a0_ref.shape[0]
    w = w_ref[...]
    winv = jnp.reshape(
        jnp.broadcast_to(w[:, :4, None], (rows, 4, C)), (rows, D))
    dinv = w[:, 4:5]
    gat = (a0_ref[...] + a1_ref[...]) * winv
    gcn = (b0_ref[...] + b1_ref[...]) * dinv
    y = (gat + gcn + x_ref[...]) * scale_ref[...] + shift_ref[...]
    o_ref[...] = jnp.where(y > 0, y, jnp.exp(jnp.minimum(y, 0.0)) - 1.0)


def _final(acc_a, acc_b, x, wtab, scale, shift):
    grid = N // _ROWS
    row = pl.BlockSpec((_ROWS, D), lambda i: (i, 0))
    row16 = pl.BlockSpec((_ROWS, 16), lambda i: (i, 0))
    vec = pl.BlockSpec((1, D), lambda i: (0, 0))
    return pl.pallas_call(
        _final_body,
        grid=(grid,),
        in_specs=[row, row, row, row, row, row16, vec, vec],
        out_specs=row,
        out_shape=jax.ShapeDtypeStruct((N, D), jnp.float32),
    )(acc_a[0, :N], acc_a[1, :N], acc_b[0, :N], acc_b[1, :N], x,
      wtab[:N], scale.reshape(1, D), shift.reshape(1, D))


def kernel(x, edge_index, W_l, W_r, att, b_gat, W_gcn, b_gcn, gamma, beta):
    ne = E + N
    loops = jnp.arange(N, dtype=jnp.int32)
    src = jnp.concatenate([edge_index[0].astype(jnp.int32), loops,
                           jnp.zeros((_EP - ne,), jnp.int32)])
    dst = jnp.concatenate([edge_index[1].astype(jnp.int32), loops,
                           jnp.full((_EP - ne,), N, jnp.int32)])

    xl, xr, xw = _projections(x, W_l, W_r, W_gcn)
    zpad = jnp.zeros((_NPAD - N, D), jnp.float32)
    xr_pad = jnp.concatenate([xr, zpad])

    alpha_flat, tmax = _pass_a(src, dst, xl, xr_pad, att.reshape(H * C))
    gmax16 = jnp.full((16,), jnp.max(tmax), jnp.float32)

    paytpl = jnp.zeros((_BB, D), jnp.float32).at[:, 4].set(1.0)
    zrows = jnp.zeros((_TROWS, D), jnp.float32)
    den = _pass_b(dst, alpha_flat, gmax16, paytpl, zrows)

    xw_tab = jnp.concatenate([xw, jnp.zeros((_NTAB - N, D), jnp.float32)])
    wtab, xws_tab = _wtab(den, xw_tab)
    xws = xws_tab[:_NPAD]

    acc_a = _pass_c1(src, dst, alpha_flat, gmax16, xl, zrows)
    acc_b = _pass_c2(src, dst, xws, zrows)

    scale = gamma * jax.lax.rsqrt(jnp.float32(1.0 + BN_EPS))
    shift = (b_gat + b_gcn) * scale + beta
    return _final(acc_a, acc_b, x, wtab, scale, shift)


# speedup vs baseline: 1.3029x; 1.0327x over previous
import jax
import jax.numpy as jnp
from jax import lax
from jax.experimental import pallas as pl
from jax.experimental.pallas import tpu as pltpu
from jax.experimental.pallas import tpu_sc as plsc

N = 10000
E = 320000
D = 128
H = 4
C = 32
NEG_SLOPE = 0.2
BN_EPS = 1e-5

_ROWS = 400
_NW = 32
_EP = 331776
_BAA = 128
_NBA = _EP // (_NW * _BAA)
_BC = 128
_NBC = _EP // (_NW * _BC)
_BB = 128
_NBB = _EP // (_NW * _BB)
_NPAD = N + 16
_NTAB = 10240
_TROWS = _NTAB // 16
_SC_MESH = dict(core_axis_name="c", subcore_axis_name="s")
_SC_PARAMS = pltpu.CompilerParams(needs_layout_passes=False)


def _pass_a_body(src_hbm, dst_hbm, xl_hbm, xr_hbm, att_hbm,
                 alpha_hbm, tmax_hbm,
                 att_v, sidx, didx, xlrows, xrrows, abuf, sem1, sem2):
    wid = lax.axis_index("s") * 2 + lax.axis_index("c")
    pltpu.sync_copy(att_hbm, att_v)
    lanes = lax.iota(jnp.int32, 16)
    rbase = lanes >> 2
    cbase = (lanes & 3) << 5
    att_c = [plsc.load_gather(att_v, [cbase + c]) for c in range(C)]

    def batch_body(b, mx):
        base = (wid * _NBA + b) * _BAA
        pltpu.sync_copy(src_hbm.at[pl.ds(base, _BAA)], sidx)
        pltpu.sync_copy(dst_hbm.at[pl.ds(base, _BAA)], didx)
        cp1 = pltpu.async_copy(xl_hbm.at[sidx], xlrows, sem1)
        cp2 = pltpu.async_copy(xr_hbm.at[didx], xrrows, sem2)
        cp1.wait()
        cp2.wait()

        def group_body(g, mx):
            row = rbase + g * 4
            acc = jnp.zeros((16,), jnp.float32)
            for c in range(C):
                col = cbase + c
                v = (plsc.load_gather(xlrows, [row, col])
                     + plsc.load_gather(xrrows, [row, col]))
                t = jnp.maximum(v, v * NEG_SLOPE)
                acc = acc + t * att_c[c]
            abuf[pl.ds(g * 16, 16)] = acc
            return jnp.maximum(mx, acc)

        mx = lax.fori_loop(0, _BAA // 4, group_body, mx)
        pltpu.sync_copy(abuf, alpha_hbm.at[pl.ds(base * 4, _BAA * 4)])
        return mx

    mx = lax.fori_loop(0, _NBA, batch_body,
                       jnp.full((16,), -3e38, jnp.float32))
    abuf[pl.ds(0, 16)] = mx
    pltpu.sync_copy(abuf.at[pl.ds(0, 16)], tmax_hbm.at[pl.ds(wid * 16, 16)])


def _pass_a(src, dst, xl, xr_pad, att_flat):
    return pl.kernel(
        _pass_a_body,
        out_type=[
            jax.ShapeDtypeStruct((_EP * 4,), jnp.float32),
            jax.ShapeDtypeStruct((_NW * 16,), jnp.float32),
        ],
        mesh=plsc.VectorSubcoreMesh(**_SC_MESH),
        compiler_params=_SC_PARAMS,
        scratch_types=[
            pltpu.VMEM((H * C,), jnp.float32),
            pltpu.VMEM((_BAA,), jnp.int32),
            pltpu.VMEM((_BAA,), jnp.int32),
            pltpu.VMEM((_BAA, D), jnp.float32),
            pltpu.VMEM((_BAA, D), jnp.float32),
            pltpu.VMEM((_BAA * 4,), jnp.float32),
            pltpu.SemaphoreType.DMA,
            pltpu.SemaphoreType.DMA,
        ],
    )(src, dst, xl, xr_pad, att_flat)


def _pass_b_body(dst_hbm, alpha_hbm, gmax_hbm, paytpl_hbm, zrows_hbm,
                 den_hbm,
                 gv, didx, abuf, pay, den_sh, sem):
    cid = lax.axis_index("c")
    sid = lax.axis_index("s")
    wid = sid * 2 + cid
    pltpu.sync_copy(gmax_hbm, gv)
    pltpu.sync_copy(paytpl_hbm, pay)
    pltpu.sync_copy(zrows_hbm, den_sh.at[pl.ds(sid * _TROWS, _TROWS)])
    plsc.subcore_barrier()
    lanes = lax.iota(jnp.int32, 16)
    rbase = lanes >> 2
    cols = lanes & 3
    g = gv[...]

    def batch_body(b, _):
        base = (wid * _NBB + b) * _BB
        pltpu.sync_copy(dst_hbm.at[pl.ds(base, _BB)], didx)
        pltpu.sync_copy(alpha_hbm.at[pl.ds(base * 4, _BB * 4)], abuf)
        for j in range(_BB // 4):
            a = abuf[pl.ds(j * 16, 16)]
            p = jnp.exp(a - g)
            plsc.store_scatter(pay, [rbase + 4 * j, cols], p)
        pltpu.sync_copy(pay, den_sh.at[didx], add=True)
        return 0

    lax.fori_loop(0, _NBB, batch_body, 0)
    plsc.subcore_barrier()
    pltpu.sync_copy(den_sh.at[pl.ds(sid * _TROWS, _TROWS)],
                    den_hbm.at[cid, pl.ds(sid * _TROWS, _TROWS)])


def _pass_b(dst, alpha_flat, gmax16, paytpl, zrows):
    return pl.kernel(
        _pass_b_body,
        out_type=[jax.ShapeDtypeStruct((2, _NTAB, D), jnp.float32)],
        mesh=plsc.VectorSubcoreMesh(**_SC_MESH),
        compiler_params=_SC_PARAMS,
        scratch_types=[
            pltpu.VMEM((16,), jnp.float32),
            pltpu.VMEM((_BB,), jnp.int32),
            pltpu.VMEM((_BB * 4,), jnp.float32),
            pltpu.VMEM((_BB, D), jnp.float32),
            pltpu.VMEM_SHARED((_NTAB, D), jnp.float32),
            pltpu.SemaphoreType.DMA,
        ],
    )(dst, alpha_flat, gmax16, paytpl, zrows)[0]


def _pass_c1_body(src_hbm, dst_hbm, alpha_hbm, gmax_hbm, xl_hbm, zrows_hbm,
                  acc_hbm,
                  gv, sidx, didx, abuf, wbuf, xlrows, contrib, acc_sh, sem):
    cid = lax.axis_index("c")
    sid = lax.axis_index("s")
    wid = sid * 2 + cid
    pltpu.sync_copy(gmax_hbm, gv)
    pltpu.sync_copy(zrows_hbm, acc_sh.at[pl.ds(sid * _TROWS, _TROWS)])
    plsc.subcore_barrier()
    g = gv[...]

    def batch_body(b, _):
        base = (wid * _NBC + b) * _BC
        pltpu.sync_copy(src_hbm.at[pl.ds(base, _BC)], sidx)
        pltpu.sync_copy(dst_hbm.at[pl.ds(base, _BC)], didx)
        pltpu.sync_copy(alpha_hbm.at[pl.ds(base * 4, _BC * 4)], abuf)
        cp = pltpu.async_copy(xl_hbm.at[sidx], xlrows, sem)
        for j in range(_BC // 4):
            a = abuf[pl.ds(j * 16, 16)]
            wbuf[pl.ds(j * 16, 16)] = jnp.exp(a - g)
        cp.wait()

        def edge_body(e, _):
            for jh in range(4):
                wsp = plsc.load_gather(
                    wbuf, [jnp.full((16,), e * 4 + jh, jnp.int32)])
                for k in range(2):
                    j = jh * 2 + k
                    xlv = xlrows[e, pl.ds(j * 16, 16)]
                    contrib[e, pl.ds(j * 16, 16)] = xlv * wsp
            return 0

        lax.fori_loop(0, _BC, edge_body, 0)
        pltpu.sync_copy(contrib, acc_sh.at[didx], add=True)
        return 0

    lax.fori_loop(0, _NBC, batch_body, 0)
    plsc.subcore_barrier()
    pltpu.sync_copy(acc_sh.at[pl.ds(sid * _TROWS, _TROWS)],
                    acc_hbm.at[cid, pl.ds(sid * _TROWS, _TROWS)])


def _pass_c1(src, dst, alpha_flat, gmax16, xl, zrows):
    return pl.kernel(
        _pass_c1_body,
        out_type=[jax.ShapeDtypeStruct((2, _NTAB, D), jnp.float32)],
        mesh=plsc.VectorSubcoreMesh(**_SC_MESH),
        compiler_params=_SC_PARAMS,
        scratch_types=[
            pltpu.VMEM((16,), jnp.float32),
            pltpu.VMEM((_BC,), jnp.int32),
            pltpu.VMEM((_BC,), jnp.int32),
            pltpu.VMEM((_BC * 4,), jnp.float32),
            pltpu.VMEM((_BC * 4,), jnp.float32),
            pltpu.VMEM((_BC, D), jnp.float32),
            pltpu.VMEM((_BC, D), jnp.float32),
            pltpu.VMEM_SHARED((_NTAB, D), jnp.float32),
            pltpu.SemaphoreType.DMA,
        ],
    )(src, dst, alpha_flat, gmax16, xl, zrows)[0]


def _pass_c2_body(src_hbm, dst_hbm, xws_hbm, zrows_hbm,
                  acc_hbm,
                  sidx_a, didx_a, sidx_b, didx_b, bufa, bufb, acc_sh,
                  sema, semb):
    cid = lax.axis_index("c")
    sid = lax.axis_index("s")
    wid = sid * 2 + cid
    pltpu.sync_copy(zrows_hbm, acc_sh.at[pl.ds(sid * _TROWS, _TROWS)])
    plsc.subcore_barrier()

    def load_idx(b, s_ref, d_ref):
        base = (wid * _NBA + b) * _BAA
        pltpu.sync_copy(src_hbm.at[pl.ds(base, _BAA)], s_ref)
        pltpu.sync_copy(dst_hbm.at[pl.ds(base, _BAA)], d_ref)

    load_idx(0, sidx_a, didx_a)
    pltpu.async_copy(xws_hbm.at[sidx_a], bufa, sema)

    def pair_body(i, _):
        load_idx(2 * i + 1, sidx_b, didx_b)
        cpb = pltpu.async_copy(xws_hbm.at[sidx_b], bufb, semb)
        pltpu.make_async_copy(xws_hbm.at[sidx_a], bufa, sema).wait()
        pltpu.sync_copy(bufa, acc_sh.at[didx_a], add=True)
        load_idx(2 * i + 2, sidx_a, didx_a)
        pltpu.async_copy(xws_hbm.at[sidx_a], bufa, sema)
        cpb.wait()
        pltpu.sync_copy(bufb, acc_sh.at[didx_b], add=True)
        return 0

    lax.fori_loop(0, (_NBA - 1) // 2, pair_body, 0)
    pltpu.make_async_copy(xws_hbm.at[sidx_a], bufa, sema).wait()
    pltpu.sync_copy(bufa, acc_sh.at[didx_a], add=True)
    plsc.subcore_barrier()
    pltpu.sync_copy(acc_sh.at[pl.ds(sid * _TROWS, _TROWS)],
                    acc_hbm.at[cid, pl.ds(sid * _TROWS, _TROWS)])


def _pass_c2(src, dst, xws, zrows):
    return pl.kernel(
        _pass_c2_body,
        out_type=[jax.ShapeDtypeStruct((2, _NTAB, D), jnp.float32)],
        mesh=plsc.VectorSubcoreMesh(**_SC_MESH),
        compiler_params=_SC_PARAMS,
        scratch_types=[
            pltpu.VMEM((_BAA,), jnp.int32),
            pltpu.VMEM((_BAA,), jnp.int32),
            pltpu.VMEM((_BAA,), jnp.int32),
            pltpu.VMEM((_BAA,), jnp.int32),
            pltpu.VMEM((_BAA, D), jnp.float32),
            pltpu.VMEM((_BAA, D), jnp.float32),
            pltpu.VMEM_SHARED((_NTAB, D), jnp.float32),
            pltpu.SemaphoreType.DMA,
            pltpu.SemaphoreType.DMA,
        ],
    )(src, dst, xws, zrows)[0]


def _wtab_body(d0_ref, d1_ref, xw_ref, w_ref, xws_ref):
    d = d0_ref[...] + d1_ref[...]
    invd = 1.0 / (d[:, :4] + 1e-16)
    deg = d[:, 4:5]
    dinv = jnp.where(deg > 0, jax.lax.rsqrt(jnp.maximum(deg, 1e-30)), 0.0)
    w_ref[...] = jnp.concatenate(
        [invd, dinv, jnp.zeros((d.shape[0], 11), jnp.float32)], axis=1)
    xws_ref[...] = xw_ref[...] * dinv


def _wtab(den, xw_pad):
    blk = 640
    row16 = pl.BlockSpec((blk, 16), lambda i: (i, 0))
    rowd = pl.BlockSpec((blk, D), lambda i: (i, 0))
    return pl.pallas_call(
        _wtab_body,
        grid=(_NTAB // blk,),
        in_specs=[pl.BlockSpec((blk, D), lambda i: (i, 0)),
                  pl.BlockSpec((blk, D), lambda i: (i, 0)),
                  rowd],
        out_specs=[row16, rowd],
        out_shape=[jax.ShapeDtypeStruct((_NTAB, 16), jnp.float32),
                   jax.ShapeDtypeStruct((_NTAB, D), jnp.float32)],
    )(den[0], den[1], xw_pad)


def _proj_body(x_ref, w_ref, xl_ref, xr_ref, xw_ref):
    y = jnp.dot(x_ref[...], w_ref[...], preferred_element_type=jnp.float32)
    xl_ref[...] = y[:, :D]
    xr_ref[...] = y[:, D:2 * D]
    xw_ref[...] = y[:, 2 * D:]


def _projections(x, W_l, W_r, W_gcn):
    wcat = jnp.concatenate([W_l, W_r, W_gcn], axis=1)
    grid = N // _ROWS
    row = pl.BlockSpec((_ROWS, D), lambda i: (i, 0))
    return pl.pallas_call(
        _proj_body,
        grid=(grid,),
        in_specs=[row, pl.BlockSpec((D, 3 * D), lambda i: (0, 0))],
        out_specs=[row, row, row],
        out_shape=[jax.ShapeDtypeStruct((N, D), jnp.float32)] * 3,
    )(x, wcat)


def _final_body(a0_ref, a1_ref, b0_ref, b1_ref, x_ref, w_ref,
                scale_ref, shift_ref, o_ref):
    rows = a0_ref.shape[0]
    w = w_ref[...]
    winv = jnp.reshape(
        jnp.broadcast_to(w[:, :4, None], (rows, 4, C)), (rows, D))
    dinv = w[:, 4:5]
    gat = (a0_ref[...] + a1_ref[...]) * winv
    gcn = (b0_ref[...] + b1_ref[...]) * dinv
    y = (gat + gcn + x_ref[...]) * scale_ref[...] + shift_ref[...]
    o_ref[...] = jnp.where(y > 0, y, jnp.exp(jnp.minimum(y, 0.0)) - 1.0)


def _final(acc_a, acc_b, x, wtab, scale, shift):
    grid = N // _ROWS
    row = pl.BlockSpec((_ROWS, D), lambda i: (i, 0))
    row16 = pl.BlockSpec((_ROWS, 16), lambda i: (i, 0))
    vec = pl.BlockSpec((1, D), lambda i: (0, 0))
    return pl.pallas_call(
        _final_body,
        grid=(grid,),
        in_specs=[row, row, row, row, row, row16, vec, vec],
        out_specs=row,
        out_shape=jax.ShapeDtypeStruct((N, D), jnp.float32),
    )(acc_a[0, :N], acc_a[1, :N], acc_b[0, :N], acc_b[1, :N], x,
      wtab[:N], scale.reshape(1, D), shift.reshape(1, D))


def kernel(x, edge_index, W_l, W_r, att, b_gat, W_gcn, b_gcn, gamma, beta):
    ne = E + N
    loops = jnp.arange(N, dtype=jnp.int32)
    src = jnp.concatenate([edge_index[0].astype(jnp.int32), loops,
                           jnp.zeros((_EP - ne,), jnp.int32)])
    dst = jnp.concatenate([edge_index[1].astype(jnp.int32), loops,
                           jnp.full((_EP - ne,), N, jnp.int32)])

    xl, xr, xw = _projections(x, W_l, W_r, W_gcn)
    zpad = jnp.zeros((_NPAD - N, D), jnp.float32)
    xr_pad = jnp.concatenate([xr, zpad])

    alpha_flat, tmax = _pass_a(src, dst, xl, xr_pad, att.reshape(H * C))
    gmax16 = jnp.full((16,), jnp.max(tmax), jnp.float32)

    paytpl = jnp.zeros((_BB, D), jnp.float32).at[:, 4].set(1.0)
    zrows = jnp.zeros((_TROWS, D), jnp.float32)
    den = _pass_b(dst, alpha_flat, gmax16, paytpl, zrows)

    xw_tab = jnp.concatenate([xw, jnp.zeros((_NTAB - N, D), jnp.float32)])
    wtab, xws_tab = _wtab(den, xw_tab)
    xws = xws_tab[:_NPAD]

    acc_a = _pass_c1(src, dst, alpha_flat, gmax16, xl, zrows)
    acc_b = _pass_c2(src, dst, xws, zrows)

    scale = gamma * jax.lax.rsqrt(jnp.float32(1.0 + BN_EPS))
    shift = (b_gat + b_gcn) * scale + beta
    return _final(acc_a, acc_b, x, wtab, scale, shift)
